# baseline (speedup 1.0000x reference)
import functools

import jax
import jax.numpy as jnp
from jax import lax
from jax.experimental import pallas as pl
from jax.experimental.pallas import tpu as pltpu
from jax.experimental.pallas import tpu_sc as plsc

BATCH = 16384
EMBED = 32
NC = 2
NS = 16
NW = NC * NS
BPW = BATCH // NW
RB = EMBED // 8
CPW = BPW // 128
NB = 2
NBAT = BPW // NB
L = 16
IDXPAD = BPW + L


@jax.jit
def _embed_lookup(uid_idx, iid_idx, utab_t, itab_t):
  mesh = plsc.VectorSubcoreMesh(core_axis_name="c", subcore_axis_name="s",
                                num_cores=NC, num_subcores=NS)

  @functools.partial(
      pl.kernel,
      out_type=(
          jax.ShapeDtypeStruct((RB, BATCH // 128, 8, 128), jnp.float32),
          jax.ShapeDtypeStruct((RB, BATCH // 128, 8, 128), jnp.float32),
      ),
      mesh=mesh,
      compiler_params=pltpu.CompilerParams(needs_layout_passes=False),
      scratch_types=[
          pltpu.VMEM((IDXPAD,), jnp.int32),
          pltpu.VMEM((IDXPAD,), jnp.int32),
          pltpu.VMEM((NB, EMBED, 128), jnp.float32),
          pltpu.VMEM((NB, EMBED, 128), jnp.float32),
          pltpu.VMEM((RB, CPW, 8, 128), jnp.float32),
          pltpu.VMEM((RB, CPW, 8, 128), jnp.float32),
      ] + [pltpu.SemaphoreType.DMA] * (2 * NB),
  )
  def body(uidx_hbm, iidx_hbm, utab_hbm, itab_hbm, uout_hbm, iout_hbm,
           uidx_s, iidx_s, ublk_v, iblk_v, uout_v, iout_v, *sems):
    usems = sems[:NB]
    isems = sems[NB:]
    w = lax.axis_index("s") * NC + lax.axis_index("c")
    base = w * BPW
    pltpu.sync_copy(uidx_hbm.at[pl.ds(base, BPW)],
                    uidx_s.at[pl.ds(0, BPW)])
    pltpu.sync_copy(iidx_hbm.at[pl.ds(base, BPW)],
                    iidx_s.at[pl.ds(0, BPW)])

    dvec0 = lax.iota(jnp.int32, L)
    dvec1 = dvec0 + L
    rvec0 = lax.bitwise_and(dvec0, 7)
    rvec1 = lax.bitwise_and(dvec1, 7)
    Rvec0 = lax.shift_right_logical(dvec0, 3)
    Rvec1 = lax.shift_right_logical(dvec1, 3)

    def issue(tab_hbm, blk_v, semv, b, idx):
      blk = lax.shift_right_logical(idx, 7)
      off = pl.multiple_of(blk * 128, 128)
      return pltpu.async_copy(tab_hbm.at[:, pl.ds(off, 128)], blk_v.at[b],
                              semv[b])

    uvec0 = uidx_s[pl.ds(0, L)]
    ivec0 = iidx_s[pl.ds(0, L)]
    for b in range(NB):
      issue(utab_hbm, ublk_v, usems, b, uvec0[b])
      issue(itab_hbm, iblk_v, isems, b, ivec0[b])

    def phase(tab_hbm, idx_s, blk_v, semv, out_v, t):
      idxv = idx_s[pl.ds(t * NB, L)]
      for b in range(NB):
        k = t * NB + b
        pltpu.make_async_copy(
            tab_hbm.at[:, pl.ds(0, 128)], blk_v.at[b], semv[b]).wait()
        idx = idxv[b]
        col = lax.bitwise_and(idx, 127)
        colv = jnp.full((L,), col, jnp.int32)
        kc = jnp.full((L,), lax.shift_right_logical(k, 7), jnp.int32)
        km = jnp.full((L,), lax.bitwise_and(k, 127), jnp.int32)
        v0 = plsc.load_gather(blk_v.at[b], [dvec0, colv])
        v1 = plsc.load_gather(blk_v.at[b], [dvec1, colv])
        plsc.store_scatter(out_v, [Rvec0, kc, rvec0, km], v0)
        plsc.store_scatter(out_v, [Rvec1, kc, rvec1, km], v1)

        @pl.when(t < NBAT - 1)
        def _():
          issue(tab_hbm, blk_v, semv, b, idxv[NB + b])

    def grp(t, _):
      phase(utab_hbm, uidx_s, ublk_v, usems, uout_v, t)
      phase(itab_hbm, iidx_s, iblk_v, isems, iout_v, t)
      return _

    lax.fori_loop(0, NBAT, grp, None)

    pltpu.sync_copy(uout_v, uout_hbm.at[:, pl.ds(w * CPW, CPW)])
    pltpu.sync_copy(iout_v, iout_hbm.at[:, pl.ds(w * CPW, CPW)])

  uout, iout = body(uid_idx, iid_idx, utab_t, itab_t)
  uout = uout.transpose(1, 3, 0, 2).reshape(BATCH, EMBED)
  iout = iout.transpose(1, 3, 0, 2).reshape(BATCH, EMBED)
  return uout, iout


def kernel(uid_input, iid_input, uid_table, iid_table):
  uid_idx = uid_input.astype(jnp.int32)
  iid_idx = iid_input.astype(jnp.int32)
  return _embed_lookup(uid_idx, iid_idx, uid_table.T, iid_table.T)


# speedup vs baseline: 1.3035x; 1.3035x over previous
import functools

import jax
import jax.numpy as jnp
from jax import lax
from jax.experimental import pallas as pl
from jax.experimental.pallas import tpu as pltpu
from jax.experimental.pallas import tpu_sc as plsc

BATCH = 16384
EMBED = 32
NC = 2
NS = 16
NW = NC * NS
BPW = BATCH // NW
RB = EMBED // 8
CPW = BPW // 128
NB = 4
NBAT = BPW // NB
L = 16
IDXPAD = BPW + L


@jax.jit
def _embed_lookup(uid_idx, iid_idx, utab_t, itab_t):
  mesh = plsc.VectorSubcoreMesh(core_axis_name="c", subcore_axis_name="s",
                                num_cores=NC, num_subcores=NS)

  @functools.partial(
      pl.kernel,
      out_type=(
          jax.ShapeDtypeStruct((RB, BATCH // 128, 8, 128), jnp.float32),
          jax.ShapeDtypeStruct((RB, BATCH // 128, 8, 128), jnp.float32),
      ),
      mesh=mesh,
      compiler_params=pltpu.CompilerParams(needs_layout_passes=False),
      scratch_types=[
          pltpu.VMEM((IDXPAD,), jnp.int32),
          pltpu.VMEM((IDXPAD,), jnp.int32),
          pltpu.VMEM((NB, EMBED, 128), jnp.float32),
          pltpu.VMEM((NB, EMBED, 128), jnp.float32),
          pltpu.VMEM((RB, CPW, 8, 128), jnp.float32),
          pltpu.VMEM((RB, CPW, 8, 128), jnp.float32),
      ] + [pltpu.SemaphoreType.DMA] * (2 * NB),
  )
  def body(uidx_hbm, iidx_hbm, utab_hbm, itab_hbm, uout_hbm, iout_hbm,
           uidx_s, iidx_s, ublk_v, iblk_v, uout_v, iout_v, *sems):
    usems = sems[:NB]
    isems = sems[NB:]
    w = lax.axis_index("s") * NC + lax.axis_index("c")
    base = w * BPW
    pltpu.sync_copy(uidx_hbm.at[pl.ds(base, BPW)],
                    uidx_s.at[pl.ds(0, BPW)])
    pltpu.sync_copy(iidx_hbm.at[pl.ds(base, BPW)],
                    iidx_s.at[pl.ds(0, BPW)])

    dvec0 = lax.iota(jnp.int32, L)
    dvec1 = dvec0 + L
    rvec0 = lax.bitwise_and(dvec0, 7)
    rvec1 = lax.bitwise_and(dvec1, 7)
    Rvec0 = lax.shift_right_logical(dvec0, 3)
    Rvec1 = lax.shift_right_logical(dvec1, 3)

    def issue(tab_hbm, blk_v, semv, b, idx):
      blk = lax.shift_right_logical(idx, 7)
      off = pl.multiple_of(blk * 128, 128)
      return pltpu.async_copy(tab_hbm.at[:, pl.ds(off, 128)], blk_v.at[b],
                              semv[b])

    uvec0 = uidx_s[pl.ds(0, L)]
    ivec0 = iidx_s[pl.ds(0, L)]
    for b in range(NB):
      issue(utab_hbm, ublk_v, usems, b, uvec0[b])
      issue(itab_hbm, iblk_v, isems, b, ivec0[b])

    def phase(tab_hbm, idx_s, blk_v, semv, out_v, t):
      idxv = idx_s[pl.ds(t * NB, L)]
      for b in range(NB):
        k = t * NB + b
        pltpu.make_async_copy(
            tab_hbm.at[:, pl.ds(0, 128)], blk_v.at[b], semv[b]).wait()
        idx = idxv[b]
        col = lax.bitwise_and(idx, 127)
        colv = jnp.full((L,), col, jnp.int32)
        kc = jnp.full((L,), lax.shift_right_logical(k, 7), jnp.int32)
        km = jnp.full((L,), lax.bitwise_and(k, 127), jnp.int32)
        v0 = plsc.load_gather(blk_v.at[b], [dvec0, colv])
        v1 = plsc.load_gather(blk_v.at[b], [dvec1, colv])
        plsc.store_scatter(out_v, [Rvec0, kc, rvec0, km], v0)
        plsc.store_scatter(out_v, [Rvec1, kc, rvec1, km], v1)

        @pl.when(t < NBAT - 1)
        def _():
          issue(tab_hbm, blk_v, semv, b, idxv[NB + b])

    def grp(t, _):
      phase(utab_hbm, uidx_s, ublk_v, usems, uout_v, t)
      phase(itab_hbm, iidx_s, iblk_v, isems, iout_v, t)
      return _

    lax.fori_loop(0, NBAT, grp, None)

    pltpu.sync_copy(uout_v, uout_hbm.at[:, pl.ds(w * CPW, CPW)])
    pltpu.sync_copy(iout_v, iout_hbm.at[:, pl.ds(w * CPW, CPW)])

  uout, iout = body(uid_idx, iid_idx, utab_t, itab_t)
  uout = uout.transpose(1, 3, 0, 2).reshape(BATCH, EMBED)
  iout = iout.transpose(1, 3, 0, 2).reshape(BATCH, EMBED)
  return uout, iout


def kernel(uid_input, iid_input, uid_table, iid_table):
  uid_idx = uid_input.astype(jnp.int32)
  iid_idx = iid_input.astype(jnp.int32)
  return _embed_lookup(uid_idx, iid_idx, uid_table.T, iid_table.T)
